# baseline (speedup 1.0000x reference)
import functools

import jax
import jax.numpy as jnp
from jax import lax
from jax.experimental import pallas as pl
from jax.experimental.pallas import tpu as pltpu

N = 10000
E = 320000
HID = 128
OUT = 384


def _mm_body(x_ref, w_ref, b_ref, o_ref):
    acc = jnp.dot(x_ref[...], w_ref[...], preferred_element_type=jnp.float32)
    o_ref[...] = acc + b_ref[...]


def _mm(x, w, b, bm):
    m, k = x.shape
    nc = w.shape[1]
    grid = pl.cdiv(m, bm)
    return pl.pallas_call(
        _mm_body,
        grid=(grid,),
        in_specs=[
            pl.BlockSpec((bm, k), lambda i: (i, 0)),
            pl.BlockSpec((k, nc), lambda i: (0, 0)),
            pl.BlockSpec((1, nc), lambda i: (0, 0)),
        ],
        out_specs=pl.BlockSpec((bm, nc), lambda i: (i, 0)),
        out_shape=jax.ShapeDtypeStruct((m, nc), jnp.float32),
    )(x, w, b.reshape(1, nc))


def _mlp2_body(x_ref, w1_ref, b1_ref, w2_ref, b2_ref, o_ref):
    h = jnp.dot(x_ref[...], w1_ref[...], preferred_element_type=jnp.float32)
    h = jnp.maximum(h + b1_ref[...], 0.0)
    o_ref[...] = (
        jnp.dot(h, w2_ref[...], preferred_element_type=jnp.float32) + b2_ref[...]
    )


def _mlp2(x, w1, b1, w2, b2, bm):
    m, k = x.shape
    h = w1.shape[1]
    nc = w2.shape[1]
    grid = pl.cdiv(m, bm)
    return pl.pallas_call(
        _mlp2_body,
        grid=(grid,),
        in_specs=[
            pl.BlockSpec((bm, k), lambda i: (i, 0)),
            pl.BlockSpec((k, h), lambda i: (0, 0)),
            pl.BlockSpec((1, h), lambda i: (0, 0)),
            pl.BlockSpec((h, nc), lambda i: (0, 0)),
            pl.BlockSpec((1, nc), lambda i: (0, 0)),
        ],
        out_specs=pl.BlockSpec((bm, nc), lambda i: (i, 0)),
        out_shape=jax.ShapeDtypeStruct((m, nc), jnp.float32),
    )(x, w1, b1.reshape(1, h), w2, b2.reshape(1, nc))


def _gmax_body(x1_ref, x2_ref, x3_ref, o_ref):
    o_ref[0, :] = jnp.max(x1_ref[...], axis=0)
    o_ref[1, :] = jnp.max(x2_ref[...], axis=0)
    o_ref[2, :] = jnp.max(x3_ref[...], axis=0)


def _gmax(x1, x2, x3):
    return pl.pallas_call(
        _gmax_body,
        out_shape=jax.ShapeDtypeStruct((3, HID), jnp.float32),
    )(x1, x2, x3)


def _final_body(x_ref, g_ref, w1a_ref, w1b_ref, b1_ref, w2_ref, b2_ref, o_ref):
    h = jnp.dot(x_ref[...], w1a_ref[...], preferred_element_type=jnp.float32)
    h = h + jnp.dot(g_ref[...], w1b_ref[...], preferred_element_type=jnp.float32)
    h = jnp.maximum(h + b1_ref[...], 0.0)
    o_ref[...] = (
        jnp.dot(h, w2_ref[...], preferred_element_type=jnp.float32) + b2_ref[...]
    )


def _final_mlp(x123, g, p, bm):
    m = x123.shape[0]
    k = x123.shape[1]
    w1a = p["W1"][:k]
    w1b = p["W1"][k:]
    grid = pl.cdiv(m, bm)
    return pl.pallas_call(
        _final_body,
        grid=(grid,),
        in_specs=[
            pl.BlockSpec((bm, k), lambda i: (i, 0)),
            pl.BlockSpec((1, OUT), lambda i: (0, 0)),
            pl.BlockSpec((k, OUT), lambda i: (0, 0)),
            pl.BlockSpec((OUT, OUT), lambda i: (0, 0)),
            pl.BlockSpec((1, OUT), lambda i: (0, 0)),
            pl.BlockSpec((OUT, OUT), lambda i: (0, 0)),
            pl.BlockSpec((1, OUT), lambda i: (0, 0)),
        ],
        out_specs=pl.BlockSpec((bm, OUT), lambda i: (i, 0)),
        out_shape=jax.ShapeDtypeStruct((m, OUT), jnp.float32),
    )(
        x123,
        g.reshape(1, 3 * HID),
        w1a,
        w1b,
        p["b1"].reshape(1, OUT),
        p["W2"],
        p["b2"].reshape(1, OUT),
    )


def _propagate(a, b, src, dst, w2, b2):
    h = jnp.maximum(a[dst] + b[src], 0.0)
    msg = _mm(h, w2, jnp.zeros((HID,), jnp.float32), 4000)
    agg = jax.ops.segment_max(msg, dst, num_segments=N)
    return jnp.where(jnp.isfinite(agg), agg + b2, 0.0)


def _conv(p, x, mei_src, mei_dst, gei_src, gei_dst):
    din = x.shape[1]
    wm1 = p["mesh"]["W1"]
    wg1 = p["geo"]["W1"]
    wcat = jnp.concatenate(
        [wm1[:din] - wm1[din:], wm1[din:], wg1[:din] - wg1[din:], wg1[din:]],
        axis=1,
    )
    zeros = jnp.zeros((HID,), jnp.float32)
    bcat = jnp.concatenate([p["mesh"]["b1"], zeros, p["geo"]["b1"], zeros])
    if din % 8 != 0:
        pad = 8 - din % 8
        x = jnp.pad(x, ((0, 0), (0, pad)))
        wcat = jnp.pad(wcat, ((0, pad), (0, 0)))
    ab = _mm(x, wcat, bcat, 2000)
    a_m, b_m = ab[:, :HID], ab[:, HID : 2 * HID]
    a_g, b_g = ab[:, 2 * HID : 3 * HID], ab[:, 3 * HID :]
    xm = _propagate(a_m, b_m, mei_src, mei_dst, p["mesh"]["W2"], p["mesh"]["b2"])
    xg = _propagate(a_g, b_g, gei_src, gei_dst, p["geo"]["W2"], p["geo"]["b2"])
    cat = jnp.concatenate([xm, xg], axis=1)
    c = p["combine"]
    return _mlp2(cat, c["W1"], c["b1"], c["W2"], c["b2"], 2000)


def kernel(x, params, edge_index, geo_index):
    m_src, m_dst = edge_index[0], edge_index[1]
    g_src, g_dst = geo_index[0], geo_index[1]
    x1 = _conv(params["conv1"], x, m_src, m_dst, g_src, g_dst)
    x2 = _conv(params["conv2"], x1, m_src, m_dst, g_src, g_dst)
    x3 = _conv(params["conv3"], x2, m_src, m_dst, g_src, g_dst)
    g3 = _gmax(x1, x2, x3)
    x123 = jnp.concatenate([x1, x2, x3], axis=1)
    return _final_mlp(x123, g3, params["final"], 2000)
